# baseline (speedup 1.0000x reference)
import functools

import jax
import jax.numpy as jnp
from jax import lax
from jax.experimental import pallas as pl
from jax.experimental.pallas import tpu as pltpu
from jax.experimental.pallas import tpu_sc as plsc

import numpy as _np

_PERM = _np.arange(256)
for _g in range(8):
    for _p in range(32):
        _PERM[32 * _g + _p] = 32 * _g + (2 * _p if _p < 16 else 2 * _p - 31)

N = 10000
E = 160000
NP = 10240
EP = 163840
NC, NS = 2, 16
NW = NC * NS
EPT = EP // NW
BR = 512

_mesh = plsc.VectorSubcoreMesh(core_axis_name="c", subcore_axis_name="s")
_sc_params = pltpu.CompilerParams(needs_layout_passes=False,
                                  use_tc_tiling_on_sc=False)

_Z16 = functools.partial(jnp.zeros, (16,), jnp.float32)


@functools.partial(
    pl.kernel, mesh=_mesh, compiler_params=_sc_params,
    out_type=jax.ShapeDtypeStruct((NC, 2, NP), jnp.float32),
    scratch_types=[
        pltpu.VMEM((NP,), jnp.float32),
        pltpu.VMEM((NP,), jnp.float32),
        pltpu.VMEM((EPT,), jnp.int32),
        pltpu.VMEM((EPT,), jnp.int32),
        pltpu.VMEM((NP // NS,), jnp.float32),
        pltpu.VMEM((NP // NS,), jnp.float32),
        pltpu.VMEM((NP // NS,), jnp.float32),
        pltpu.VMEM_SHARED((NS, 2, NP), jnp.float32),
    ],
)
def _deg(srcf, dstf, out_hbm, cs, cd, srcv, dstv, tmp, accs, accd, shared):
    c = lax.axis_index("c")
    s = lax.axis_index("s")
    w = c * NS + s
    zero16 = _Z16()
    ones16 = jnp.full((16,), 1.0, jnp.float32)

    def z(i, _):
        cs[pl.ds(i * 16, 16)] = zero16
        cd[pl.ds(i * 16, 16)] = zero16
        return 0
    lax.fori_loop(0, NP // 16, z, 0)

    pltpu.sync_copy(srcf.at[pl.ds(w * EPT, EPT)], srcv)
    pltpu.sync_copy(dstf.at[pl.ds(w * EPT, EPT)], dstv)

    def acc_edges(i, _):
        plsc.addupdate_scatter(cs, [srcv[pl.ds(i * 16, 16)]], ones16)
        plsc.addupdate_scatter(cd, [dstv[pl.ds(i * 16, 16)]], ones16)
        return 0
    lax.fori_loop(0, EPT // 16, acc_edges, 0)

    pltpu.sync_copy(cs, shared.at[s, 0])
    pltpu.sync_copy(cd, shared.at[s, 1])
    plsc.subcore_barrier()

    nrows = NP // NS
    base = s * nrows

    def z2(i, _):
        accs[pl.ds(i * 16, 16)] = zero16
        accd[pl.ds(i * 16, 16)] = zero16
        return 0
    lax.fori_loop(0, nrows // 16, z2, 0)

    def red(t, _):
        pltpu.sync_copy(shared.at[t, 0, pl.ds(base, nrows)], tmp)

        def a1(i, _):
            accs[pl.ds(i * 16, 16)] = accs[pl.ds(i * 16, 16)] + tmp[pl.ds(i * 16, 16)]
            return 0
        lax.fori_loop(0, nrows // 16, a1, 0)
        pltpu.sync_copy(shared.at[t, 1, pl.ds(base, nrows)], tmp)

        def a2(i, _):
            accd[pl.ds(i * 16, 16)] = accd[pl.ds(i * 16, 16)] + tmp[pl.ds(i * 16, 16)]
            return 0
        lax.fori_loop(0, nrows // 16, a2, 0)
        return 0
    lax.fori_loop(0, NS, red, 0)

    pltpu.sync_copy(accs, out_hbm.at[c, 0, pl.ds(base, nrows)])
    pltpu.sync_copy(accd, out_hbm.at[c, 1, pl.ds(base, nrows)])


@functools.partial(
    pl.kernel, mesh=_mesh, compiler_params=_sc_params,
    out_type=jax.ShapeDtypeStruct((NC, NP, 128), jnp.float32),
    scratch_types=[
        pltpu.VMEM((80, 64), jnp.int32),
        pltpu.VMEM((80, 64), jnp.int32),
        pltpu.VMEM((4, 64, 128), jnp.bfloat16),
        pltpu.VMEM((64, 128), jnp.float32),
        pltpu.VMEM_SHARED((NP, 128), jnp.float32),
        pltpu.SemaphoreType.DMA,
        pltpu.SemaphoreType.DMA,
        pltpu.SemaphoreType.DMA,
        pltpu.SemaphoreType.DMA,
    ],
)
def _agg1(y0, y1, src3, dst3, out_hbm, srcv, dstv, rows, rowsf, acc,
          g0, g1, g2, g3):
    c = lax.axis_index("c")
    s = lax.axis_index("s")
    nch = EP // NS // 64
    half = nch // 2

    zero16 = _Z16()

    def z(i, _):
        rowsf[i // 8, pl.ds((i % 8) * 16, 16)] = zero16
        return 0
    lax.fori_loop(0, 512, z, 0)

    nrows = NP // NS

    def zacc(t, _):
        pltpu.sync_copy(rowsf, acc.at[pl.ds(s * nrows + t * 64, 64)])
        return 0
    lax.fori_loop(0, nrows // 64, zacc, 0)
    plsc.subcore_barrier()

    def run_edges(y_hbm):
        gsem = (g0, g1, g2, g3)
        nbuf = 4
        mask_hi = jnp.full((16,), -65536, jnp.int32)

        def wait_g(b):
            pltpu.make_async_copy(y_hbm.at[srcv.at[0]],
                                  rows.at[b], gsem[b]).wait()

        def convert(b):
            def cv(r, _):
                for k in range(4):
                    v = rows[b, r, pl.ds(k * 32, 32)]
                    xi = plsc.bitcast(v, jnp.int32)
                    lo = plsc.bitcast(xi << 16, jnp.float32)
                    hi = plsc.bitcast(xi & mask_hi, jnp.float32)
                    rowsf[r, pl.ds(k * 32, 16)] = lo
                    rowsf[r, pl.ds(k * 32 + 16, 16)] = hi
                return 0
            lax.fori_loop(0, 64, cv, 0)

        for h in range(2):
            base = s * nch + h * half
            pltpu.sync_copy(src3.at[pl.ds(base, half)], srcv)
            pltpu.sync_copy(dst3.at[pl.ds(base, half)], dstv)
            for b in range(nbuf):
                pltpu.async_copy(y_hbm.at[srcv.at[b]], rows.at[b], gsem[b])

            def step(g, _):
                for b in range(nbuf):
                    j = g * nbuf + b
                    wait_g(b)
                    convert(b)

                    @pl.when(j + nbuf < half)
                    def _():
                        pltpu.async_copy(y_hbm.at[srcv.at[j + nbuf]],
                                         rows.at[b], gsem[b])
                    pltpu.sync_copy(rowsf, acc.at[dstv.at[j]], add=True)
                return 0
            lax.fori_loop(0, half // nbuf, step, 0)

    @pl.when(c == 0)
    def _():
        run_edges(y0)

    @pl.when(c == 1)
    def _():
        run_edges(y1)

    plsc.subcore_barrier()

    def wb(t, _):
        pltpu.sync_copy(acc.at[pl.ds(s * nrows + t * 64, 64)], rowsf)
        pltpu.sync_copy(rowsf, out_hbm.at[c, pl.ds(s * nrows + t * 64, 64)])
        return 0
    lax.fori_loop(0, nrows // 64, wb, 0)


@functools.partial(
    pl.kernel, mesh=_mesh, compiler_params=_sc_params,
    out_type=jax.ShapeDtypeStruct((NC, NP, 16), jnp.float32),
    scratch_types=[
        pltpu.VMEM((EPT // 128, 128), jnp.int32),
        pltpu.VMEM((EPT // 128, 128), jnp.int32),
        pltpu.VMEM((4, 128, 16), jnp.float32),
        pltpu.VMEM_SHARED((NP, 16), jnp.float32),
        pltpu.SemaphoreType.DMA,
        pltpu.SemaphoreType.DMA,
        pltpu.SemaphoreType.DMA,
        pltpu.SemaphoreType.DMA,
        pltpu.SemaphoreType.DMA,
        pltpu.SemaphoreType.DMA,
        pltpu.SemaphoreType.DMA,
        pltpu.SemaphoreType.DMA,
    ],
)
def _agg2(q, src2, dst2, out_hbm, srcv, dstv, rows, acc,
          g0, g1, g2, g3, s0, s1, s2, s3):
    c = lax.axis_index("c")
    s = lax.axis_index("s")
    w = c * NS + s
    nch = EPT // 128
    pltpu.sync_copy(src2.at[pl.ds(w * nch, nch)], srcv)
    pltpu.sync_copy(dst2.at[pl.ds(w * nch, nch)], dstv)

    zero16 = _Z16()

    def z(j, _):
        rows[0, j, pl.ds(0, 16)] = zero16
        return 0
    lax.fori_loop(0, 128, z, 0)

    nrows = NP // NS

    def zacc(t, _):
        pltpu.sync_copy(rows.at[0], acc.at[pl.ds(s * nrows + t * 128, 128)])
        return 0
    lax.fori_loop(0, nrows // 128, zacc, 0)
    plsc.subcore_barrier()

    gsem = (g0, g1, g2, g3)
    ssem = (s0, s1, s2, s3)
    nbuf = 4

    def wait_g(b):
        pltpu.make_async_copy(q.at[srcv.at[0]], rows.at[b], gsem[b]).wait()

    def wait_s(b):
        pltpu.make_async_copy(rows.at[b], acc.at[pl.ds(0, 128)],
                              ssem[b]).wait()

    for b in range(nbuf):
        pltpu.async_copy(q.at[srcv.at[b]], rows.at[b], gsem[b])

    def step(g, _):
        for b in range(nbuf):
            j = g * nbuf + b
            wait_g(b)
            pltpu.sync_copy(rows.at[b], acc.at[dstv.at[j]], add=True)

            @pl.when(j + nbuf < nch)
            def _():
                pltpu.async_copy(q.at[srcv.at[j + nbuf]], rows.at[b], gsem[b])
        return 0
    lax.fori_loop(0, nch // nbuf, step, 0)

    plsc.subcore_barrier()

    def wb(t, _):
        pltpu.sync_copy(acc.at[pl.ds(s * nrows + t * 128, 128)], rows.at[0])
        pltpu.sync_copy(rows.at[0], out_hbm.at[c, pl.ds(s * nrows + t * 128, 128)])
        return 0
    lax.fori_loop(0, nrows // 128, wb, 0)


def _k1_body(x_ref, w_ref, cnt_ref, y_ref, dinv_ref):
    cnt = cnt_ref[...]
    dinv = lax.rsqrt(jnp.maximum(cnt[0] + cnt[1], 1.0))
    dinv_ref[...] = dinv
    y = jnp.dot(x_ref[...], w_ref[...], preferred_element_type=jnp.float32)
    y_ref[0] = (y * dinv[0][:, None]).astype(jnp.bfloat16)


def _k1(x_p, W1, counts):
    return pl.pallas_call(
        _k1_body,
        grid=(2, NP // BR),
        in_specs=[
            pl.BlockSpec((BR, 256), lambda c, r: (r, 0)),
            pl.BlockSpec((256, 128), lambda c, r: (0, c)),
            pl.BlockSpec((2, 2, BR), lambda c, r: (0, 0, r)),
        ],
        out_specs=[
            pl.BlockSpec((1, BR, 128), lambda c, r: (c, r, 0)),
            pl.BlockSpec((2, BR), lambda c, r: (0, r)),
        ],
        out_shape=[
            jax.ShapeDtypeStruct((2, NP, 128), jnp.bfloat16),
            jax.ShapeDtypeStruct((2, NP), jnp.float32),
        ],
    )(x_p, W1, counts)


def _k2_body(agg_ref, dinv_ref, b1_ref, w4t_ref, w4b_ref, w2_ref, et_ref,
             b2_ref, idx_ref, q_ref):
    a = agg_ref[...]
    dinv = dinv_ref[...]
    aggf = jnp.concatenate([a[0], a[1]], axis=1)
    h = jnp.maximum(aggf * dinv[1][:, None] + b1_ref[...], 0.0)
    acc = jnp.dot(h, w4t_ref[...], preferred_element_type=jnp.float32)
    M = jnp.dot(w2_ref[...], w4b_ref[...],
                preferred_element_type=jnp.float32).reshape(4, 256, 16)
    idx = idx_ref[...]
    et = et_ref[...]
    for l in range(4):
        g = jnp.dot(et, M[l], preferred_element_type=jnp.float32)
        oh = (idx[:, l][:, None]
              == lax.broadcasted_iota(jnp.int32, (BR, 256), 1)).astype(jnp.float32)
        acc = acc + jnp.dot(oh, g, preferred_element_type=jnp.float32)
    c0 = jnp.dot(b2_ref[...], w4b_ref[...], preferred_element_type=jnp.float32)
    q_ref[...] = (acc + c0) * dinv[0][:, None]


def _k2(agg, dinv, b1, W4t, W4b, W2, et, b2, idx_p):
    return pl.pallas_call(
        _k2_body,
        grid=(NP // BR,),
        in_specs=[
            pl.BlockSpec((2, BR, 128), lambda r: (0, r, 0)),
            pl.BlockSpec((2, BR), lambda r: (0, r)),
            pl.BlockSpec((1, 256), lambda r: (0, 0)),
            pl.BlockSpec((256, 16), lambda r: (0, 0)),
            pl.BlockSpec((2, 16), lambda r: (0, 0)),
            pl.BlockSpec((1024, 2), lambda r: (0, 0)),
            pl.BlockSpec((256, 256), lambda r: (0, 0)),
            pl.BlockSpec((1, 2), lambda r: (0, 0)),
            pl.BlockSpec((BR, 4), lambda r: (r, 0)),
        ],
        out_specs=pl.BlockSpec((BR, 16), lambda r: (r, 0)),
        out_shape=jax.ShapeDtypeStruct((NP, 16), jnp.float32),
    )(agg, dinv, b1, W4t, W4b, W2, et, b2, idx_p)


def _k3_body(p_ref, dinv_ref, b4_ref, o_ref):
    p = p_ref[...]
    o_ref[...] = (p[0] + p[1]) * dinv_ref[...][1][:, None] + b4_ref[...]


def _k3(parts, dinv, b4):
    return pl.pallas_call(
        _k3_body,
        grid=(NP // BR,),
        in_specs=[
            pl.BlockSpec((2, BR, 16), lambda r: (0, r, 0)),
            pl.BlockSpec((2, BR), lambda r: (0, r)),
            pl.BlockSpec((1, 16), lambda r: (0, 0)),
        ],
        out_specs=pl.BlockSpec((BR, 16), lambda r: (r, 0)),
        out_shape=jax.ShapeDtypeStruct((NP, 16), jnp.float32),
    )(parts, dinv, b4)


def kernel(in_feat, edge_index, encoder_input_data, W1, b1, embed_table,
           W2, b2, W4, b4):
    ei = edge_index.astype(jnp.int32)
    padv = jnp.full((EP - E,), N, jnp.int32)
    src_p = jnp.concatenate([ei[0], padv])
    dst_p = jnp.concatenate([ei[1], padv])
    src2 = src_p.reshape(EP // 128, 128)
    dst2 = dst_p.reshape(EP // 128, 128)
    x_p = jnp.pad(in_feat, ((0, NP - N), (0, 0)))
    idx_p = jnp.pad(encoder_input_data.astype(jnp.int32), ((0, NP - N), (0, 0)))

    src3 = src_p.reshape(EP // 64, 64)
    dst3 = dst_p.reshape(EP // 64, 64)

    counts = _deg(src_p, dst_p)
    y3, dinv = _k1(x_p, W1, counts)
    agg = _agg1(y3[0], y3[1], src3, dst3)
    perm = jnp.asarray(_PERM)
    b1p = b1[perm].reshape(1, 256)
    w4tp = W4[:256][perm]
    q = _k2(agg, dinv, b1p, w4tp, W4[256:], W2,
            embed_table, b2.reshape(1, 2), idx_p)
    parts = _agg2(q, src2, dst2)
    out = _k3(parts, dinv, b4.reshape(1, 16))
    return out[:N]


# speedup vs baseline: 1.1554x; 1.1554x over previous
import functools

import jax
import jax.numpy as jnp
from jax import lax
from jax.experimental import pallas as pl
from jax.experimental.pallas import tpu as pltpu
from jax.experimental.pallas import tpu_sc as plsc

N = 10000
E = 160000
NP = 10240
EP = 163840
NC, NS = 2, 16
NW = NC * NS
EPT = EP // NW
BR = 512

_mesh = plsc.VectorSubcoreMesh(core_axis_name="c", subcore_axis_name="s")
_sc_params = pltpu.CompilerParams(needs_layout_passes=False,
                                  use_tc_tiling_on_sc=False)

_Z16 = functools.partial(jnp.zeros, (16,), jnp.float32)


@functools.partial(
    pl.kernel, mesh=_mesh, compiler_params=_sc_params,
    out_type=jax.ShapeDtypeStruct((NC, 2, NP), jnp.float32),
    scratch_types=[
        pltpu.VMEM((NP,), jnp.float32),
        pltpu.VMEM((NP,), jnp.float32),
        pltpu.VMEM((EPT,), jnp.int32),
        pltpu.VMEM((EPT,), jnp.int32),
        pltpu.VMEM((NP // NS,), jnp.float32),
        pltpu.VMEM((NP // NS,), jnp.float32),
        pltpu.VMEM((NP // NS,), jnp.float32),
        pltpu.VMEM_SHARED((NS, 2, NP), jnp.float32),
    ],
)
def _deg(srcf, dstf, out_hbm, cs, cd, srcv, dstv, tmp, accs, accd, shared):
    c = lax.axis_index("c")
    s = lax.axis_index("s")
    w = c * NS + s
    zero16 = _Z16()
    ones16 = jnp.full((16,), 1.0, jnp.float32)

    def z(i, _):
        cs[pl.ds(i * 16, 16)] = zero16
        cd[pl.ds(i * 16, 16)] = zero16
        return 0
    lax.fori_loop(0, NP // 16, z, 0)

    pltpu.sync_copy(srcf.at[pl.ds(w * EPT, EPT)], srcv)
    pltpu.sync_copy(dstf.at[pl.ds(w * EPT, EPT)], dstv)

    def acc_edges(i, _):
        plsc.addupdate_scatter(cs, [srcv[pl.ds(i * 16, 16)]], ones16)
        plsc.addupdate_scatter(cd, [dstv[pl.ds(i * 16, 16)]], ones16)
        return 0
    lax.fori_loop(0, EPT // 16, acc_edges, 0)

    pltpu.sync_copy(cs, shared.at[s, 0])
    pltpu.sync_copy(cd, shared.at[s, 1])
    plsc.subcore_barrier()

    nrows = NP // NS
    base = s * nrows

    def z2(i, _):
        accs[pl.ds(i * 16, 16)] = zero16
        accd[pl.ds(i * 16, 16)] = zero16
        return 0
    lax.fori_loop(0, nrows // 16, z2, 0)

    def red(t, _):
        pltpu.sync_copy(shared.at[t, 0, pl.ds(base, nrows)], tmp)

        def a1(i, _):
            accs[pl.ds(i * 16, 16)] = accs[pl.ds(i * 16, 16)] + tmp[pl.ds(i * 16, 16)]
            return 0
        lax.fori_loop(0, nrows // 16, a1, 0)
        pltpu.sync_copy(shared.at[t, 1, pl.ds(base, nrows)], tmp)

        def a2(i, _):
            accd[pl.ds(i * 16, 16)] = accd[pl.ds(i * 16, 16)] + tmp[pl.ds(i * 16, 16)]
            return 0
        lax.fori_loop(0, nrows // 16, a2, 0)
        return 0
    lax.fori_loop(0, NS, red, 0)

    pltpu.sync_copy(accs, out_hbm.at[c, 0, pl.ds(base, nrows)])
    pltpu.sync_copy(accd, out_hbm.at[c, 1, pl.ds(base, nrows)])


@functools.partial(
    pl.kernel, mesh=_mesh, compiler_params=_sc_params,
    out_type=jax.ShapeDtypeStruct((NC, NP, 128), jnp.float32),
    scratch_types=[
        pltpu.VMEM((160, 32), jnp.int32),
        pltpu.VMEM((160, 32), jnp.int32),
        pltpu.VMEM((8, 32, 128), jnp.float32),
        pltpu.VMEM_SHARED((NP, 128), jnp.float32),
        pltpu.SemaphoreType.DMA,
        pltpu.SemaphoreType.DMA,
        pltpu.SemaphoreType.DMA,
        pltpu.SemaphoreType.DMA,
        pltpu.SemaphoreType.DMA,
        pltpu.SemaphoreType.DMA,
        pltpu.SemaphoreType.DMA,
        pltpu.SemaphoreType.DMA,
    ],
)
def _agg1(y0, y1, src3, dst3, out_hbm, srcv, dstv, rows, acc,
          g0, g1, g2, g3, g4, g5, g6, g7):
    c = lax.axis_index("c")
    s = lax.axis_index("s")
    nch = EP // NS // 32
    half = nch // 2

    zero16 = _Z16()

    def z(i, _):
        rows[i // 256, (i // 8) % 32, pl.ds((i % 8) * 16, 16)] = zero16
        return 0
    lax.fori_loop(0, 512, z, 0)

    nrows = NP // NS

    def zacc(t, _):
        pltpu.sync_copy(rows.at[0], acc.at[pl.ds(s * nrows + t * 32, 32)])
        return 0
    lax.fori_loop(0, nrows // 32, zacc, 0)
    plsc.subcore_barrier()

    def run_edges(y_hbm):
        gsem = (g0, g1, g2, g3, g4, g5, g6, g7)
        nbuf = 8

        def wait_g(b):
            pltpu.make_async_copy(y_hbm.at[srcv.at[0]],
                                  rows.at[b], gsem[b]).wait()

        def wait_s(b):
            pltpu.make_async_copy(rows.at[b],
                                  acc.at[pl.ds(0, 64)], ssem[b]).wait()

        for h in range(2):
            base = s * nch + h * half
            pltpu.sync_copy(src3.at[pl.ds(base, half)], srcv)
            pltpu.sync_copy(dst3.at[pl.ds(base, half)], dstv)
            for b in range(nbuf):
                pltpu.async_copy(y_hbm.at[srcv.at[b]], rows.at[b], gsem[b])

            def step(g, _):
                for b in range(nbuf):
                    j = g * nbuf + b
                    wait_g(b)
                    pltpu.sync_copy(rows.at[b], acc.at[dstv.at[j]], add=True)

                    @pl.when(j + nbuf < half)
                    def _():
                        pltpu.async_copy(y_hbm.at[srcv.at[j + nbuf]],
                                         rows.at[b], gsem[b])
                return 0
            lax.fori_loop(0, half // nbuf, step, 0)

    @pl.when(c == 0)
    def _():
        run_edges(y0)

    @pl.when(c == 1)
    def _():
        run_edges(y1)

    plsc.subcore_barrier()

    def wb(t, _):
        pltpu.sync_copy(acc.at[pl.ds(s * nrows + t * 32, 32)], rows.at[0])
        pltpu.sync_copy(rows.at[0], out_hbm.at[c, pl.ds(s * nrows + t * 32, 32)])
        return 0
    lax.fori_loop(0, nrows // 32, wb, 0)


@functools.partial(
    pl.kernel, mesh=_mesh, compiler_params=_sc_params,
    out_type=jax.ShapeDtypeStruct((NC, NP, 16), jnp.float32),
    scratch_types=[
        pltpu.VMEM((EPT // 128, 128), jnp.int32),
        pltpu.VMEM((EPT // 128, 128), jnp.int32),
        pltpu.VMEM((4, 128, 16), jnp.float32),
        pltpu.VMEM_SHARED((NP, 16), jnp.float32),
        pltpu.SemaphoreType.DMA,
        pltpu.SemaphoreType.DMA,
        pltpu.SemaphoreType.DMA,
        pltpu.SemaphoreType.DMA,
        pltpu.SemaphoreType.DMA,
        pltpu.SemaphoreType.DMA,
        pltpu.SemaphoreType.DMA,
        pltpu.SemaphoreType.DMA,
    ],
)
def _agg2(q, src2, dst2, out_hbm, srcv, dstv, rows, acc,
          g0, g1, g2, g3, s0, s1, s2, s3):
    c = lax.axis_index("c")
    s = lax.axis_index("s")
    w = c * NS + s
    nch = EPT // 128
    pltpu.sync_copy(src2.at[pl.ds(w * nch, nch)], srcv)
    pltpu.sync_copy(dst2.at[pl.ds(w * nch, nch)], dstv)

    zero16 = _Z16()

    def z(j, _):
        rows[0, j, pl.ds(0, 16)] = zero16
        return 0
    lax.fori_loop(0, 128, z, 0)

    nrows = NP // NS

    def zacc(t, _):
        pltpu.sync_copy(rows.at[0], acc.at[pl.ds(s * nrows + t * 128, 128)])
        return 0
    lax.fori_loop(0, nrows // 128, zacc, 0)
    plsc.subcore_barrier()

    gsem = (g0, g1, g2, g3)
    ssem = (s0, s1, s2, s3)
    nbuf = 4

    def wait_g(b):
        pltpu.make_async_copy(q.at[srcv.at[0]], rows.at[b], gsem[b]).wait()

    def wait_s(b):
        pltpu.make_async_copy(rows.at[b], acc.at[pl.ds(0, 128)],
                              ssem[b]).wait()

    for b in range(nbuf):
        pltpu.async_copy(q.at[srcv.at[b]], rows.at[b], gsem[b])

    def step(g, _):
        for b in range(nbuf):
            j = g * nbuf + b
            wait_g(b)
            pltpu.sync_copy(rows.at[b], acc.at[dstv.at[j]], add=True)

            @pl.when(j + nbuf < nch)
            def _():
                pltpu.async_copy(q.at[srcv.at[j + nbuf]], rows.at[b], gsem[b])
        return 0
    lax.fori_loop(0, nch // nbuf, step, 0)

    plsc.subcore_barrier()

    def wb(t, _):
        pltpu.sync_copy(acc.at[pl.ds(s * nrows + t * 128, 128)], rows.at[0])
        pltpu.sync_copy(rows.at[0], out_hbm.at[c, pl.ds(s * nrows + t * 128, 128)])
        return 0
    lax.fori_loop(0, nrows // 128, wb, 0)


def _k1_body(x_ref, w_ref, cnt_ref, y_ref, dinv_ref):
    cnt = cnt_ref[...]
    dinv = lax.rsqrt(jnp.maximum(cnt[0] + cnt[1], 1.0))
    dinv_ref[...] = dinv
    y = jnp.dot(x_ref[...], w_ref[...], preferred_element_type=jnp.float32)
    y_ref[0] = y * dinv[0][:, None]


def _k1(x_p, W1, counts):
    return pl.pallas_call(
        _k1_body,
        grid=(2, NP // BR),
        in_specs=[
            pl.BlockSpec((BR, 256), lambda c, r: (r, 0)),
            pl.BlockSpec((256, 128), lambda c, r: (0, c)),
            pl.BlockSpec((2, 2, BR), lambda c, r: (0, 0, r)),
        ],
        out_specs=[
            pl.BlockSpec((1, BR, 128), lambda c, r: (c, r, 0)),
            pl.BlockSpec((2, BR), lambda c, r: (0, r)),
        ],
        out_shape=[
            jax.ShapeDtypeStruct((2, NP, 128), jnp.float32),
            jax.ShapeDtypeStruct((2, NP), jnp.float32),
        ],
    )(x_p, W1, counts)


def _k2_body(agg_ref, dinv_ref, b1_ref, w4t_ref, w4b_ref, w2_ref, et_ref,
             b2_ref, idx_ref, q_ref):
    a = agg_ref[...]
    dinv = dinv_ref[...]
    aggf = jnp.concatenate([a[0], a[1]], axis=1)
    h = jnp.maximum(aggf * dinv[1][:, None] + b1_ref[...], 0.0)
    acc = jnp.dot(h, w4t_ref[...], preferred_element_type=jnp.float32)
    M = jnp.dot(w2_ref[...], w4b_ref[...],
                preferred_element_type=jnp.float32).reshape(4, 256, 16)
    idx = idx_ref[...]
    et = et_ref[...]
    for l in range(4):
        g = jnp.dot(et, M[l], preferred_element_type=jnp.float32)
        oh = (idx[:, l][:, None]
              == lax.broadcasted_iota(jnp.int32, (BR, 256), 1)).astype(jnp.float32)
        acc = acc + jnp.dot(oh, g, preferred_element_type=jnp.float32)
    c0 = jnp.dot(b2_ref[...], w4b_ref[...], preferred_element_type=jnp.float32)
    q_ref[...] = (acc + c0) * dinv[0][:, None]


def _k2(agg, dinv, b1, W4t, W4b, W2, et, b2, idx_p):
    return pl.pallas_call(
        _k2_body,
        grid=(NP // BR,),
        in_specs=[
            pl.BlockSpec((2, BR, 128), lambda r: (0, r, 0)),
            pl.BlockSpec((2, BR), lambda r: (0, r)),
            pl.BlockSpec((1, 256), lambda r: (0, 0)),
            pl.BlockSpec((256, 16), lambda r: (0, 0)),
            pl.BlockSpec((2, 16), lambda r: (0, 0)),
            pl.BlockSpec((1024, 2), lambda r: (0, 0)),
            pl.BlockSpec((256, 256), lambda r: (0, 0)),
            pl.BlockSpec((1, 2), lambda r: (0, 0)),
            pl.BlockSpec((BR, 4), lambda r: (r, 0)),
        ],
        out_specs=pl.BlockSpec((BR, 16), lambda r: (r, 0)),
        out_shape=jax.ShapeDtypeStruct((NP, 16), jnp.float32),
    )(agg, dinv, b1, W4t, W4b, W2, et, b2, idx_p)


def _k3_body(p_ref, dinv_ref, b4_ref, o_ref):
    p = p_ref[...]
    o_ref[...] = (p[0] + p[1]) * dinv_ref[...][1][:, None] + b4_ref[...]


def _k3(parts, dinv, b4):
    return pl.pallas_call(
        _k3_body,
        grid=(NP // BR,),
        in_specs=[
            pl.BlockSpec((2, BR, 16), lambda r: (0, r, 0)),
            pl.BlockSpec((2, BR), lambda r: (0, r)),
            pl.BlockSpec((1, 16), lambda r: (0, 0)),
        ],
        out_specs=pl.BlockSpec((BR, 16), lambda r: (r, 0)),
        out_shape=jax.ShapeDtypeStruct((NP, 16), jnp.float32),
    )(parts, dinv, b4)


def kernel(in_feat, edge_index, encoder_input_data, W1, b1, embed_table,
           W2, b2, W4, b4):
    ei = edge_index.astype(jnp.int32)
    padv = jnp.full((EP - E,), N, jnp.int32)
    src_p = jnp.concatenate([ei[0], padv])
    dst_p = jnp.concatenate([ei[1], padv])
    src2 = src_p.reshape(EP // 128, 128)
    dst2 = dst_p.reshape(EP // 128, 128)
    x_p = jnp.pad(in_feat, ((0, NP - N), (0, 0)))
    idx_p = jnp.pad(encoder_input_data.astype(jnp.int32), ((0, NP - N), (0, 0)))

    src3 = src_p.reshape(EP // 32, 32)
    dst3 = dst_p.reshape(EP // 32, 32)

    counts = _deg(src_p, dst_p)
    y3, dinv = _k1(x_p, W1, counts)
    agg = _agg1(y3[0], y3[1], src3, dst3)
    q = _k2(agg, dinv, b1.reshape(1, 256), W4[:256], W4[256:], W2,
            embed_table, b2.reshape(1, 2), idx_p)
    parts = _agg2(q, src2, dst2)
    out = _k3(parts, dinv, b4.reshape(1, 16))
    return out[:N]


# speedup vs baseline: 1.1612x; 1.0050x over previous
import functools

import jax
import jax.numpy as jnp
from jax import lax
from jax.experimental import pallas as pl
from jax.experimental.pallas import tpu as pltpu
from jax.experimental.pallas import tpu_sc as plsc

N = 10000
E = 160000
NP = 10240
EP = 163840
NC, NS = 2, 16
NW = NC * NS
EPT = EP // NW
BR = 512

_mesh = plsc.VectorSubcoreMesh(core_axis_name="c", subcore_axis_name="s")
_sc_params = pltpu.CompilerParams(needs_layout_passes=False,
                                  use_tc_tiling_on_sc=False)

_Z16 = functools.partial(jnp.zeros, (16,), jnp.float32)


@functools.partial(
    pl.kernel, mesh=_mesh, compiler_params=_sc_params,
    out_type=jax.ShapeDtypeStruct((NC, 2, NP), jnp.float32),
    scratch_types=[
        pltpu.VMEM((NP,), jnp.float32),
        pltpu.VMEM((NP,), jnp.float32),
        pltpu.VMEM((EPT,), jnp.int32),
        pltpu.VMEM((EPT,), jnp.int32),
        pltpu.VMEM((NP // NS,), jnp.float32),
        pltpu.VMEM((NP // NS,), jnp.float32),
        pltpu.VMEM((NP // NS,), jnp.float32),
        pltpu.VMEM_SHARED((NS, 2, NP), jnp.float32),
    ],
)
def _deg(srcf, dstf, out_hbm, cs, cd, srcv, dstv, tmp, accs, accd, shared):
    c = lax.axis_index("c")
    s = lax.axis_index("s")
    w = c * NS + s
    zero16 = _Z16()
    ones16 = jnp.full((16,), 1.0, jnp.float32)

    def z(i, _):
        cs[pl.ds(i * 16, 16)] = zero16
        cd[pl.ds(i * 16, 16)] = zero16
        return 0
    lax.fori_loop(0, NP // 16, z, 0)

    pltpu.sync_copy(srcf.at[pl.ds(w * EPT, EPT)], srcv)
    pltpu.sync_copy(dstf.at[pl.ds(w * EPT, EPT)], dstv)

    def acc_edges(i, _):
        plsc.addupdate_scatter(cs, [srcv[pl.ds(i * 16, 16)]], ones16)
        plsc.addupdate_scatter(cd, [dstv[pl.ds(i * 16, 16)]], ones16)
        return 0
    lax.fori_loop(0, EPT // 16, acc_edges, 0)

    pltpu.sync_copy(cs, shared.at[s, 0])
    pltpu.sync_copy(cd, shared.at[s, 1])
    plsc.subcore_barrier()

    nrows = NP // NS
    base = s * nrows

    def z2(i, _):
        accs[pl.ds(i * 16, 16)] = zero16
        accd[pl.ds(i * 16, 16)] = zero16
        return 0
    lax.fori_loop(0, nrows // 16, z2, 0)

    def red(t, _):
        pltpu.sync_copy(shared.at[t, 0, pl.ds(base, nrows)], tmp)

        def a1(i, _):
            accs[pl.ds(i * 16, 16)] = accs[pl.ds(i * 16, 16)] + tmp[pl.ds(i * 16, 16)]
            return 0
        lax.fori_loop(0, nrows // 16, a1, 0)
        pltpu.sync_copy(shared.at[t, 1, pl.ds(base, nrows)], tmp)

        def a2(i, _):
            accd[pl.ds(i * 16, 16)] = accd[pl.ds(i * 16, 16)] + tmp[pl.ds(i * 16, 16)]
            return 0
        lax.fori_loop(0, nrows // 16, a2, 0)
        return 0
    lax.fori_loop(0, NS, red, 0)

    pltpu.sync_copy(accs, out_hbm.at[c, 0, pl.ds(base, nrows)])
    pltpu.sync_copy(accd, out_hbm.at[c, 1, pl.ds(base, nrows)])


@functools.partial(
    pl.kernel, mesh=_mesh, compiler_params=_sc_params,
    out_type=jax.ShapeDtypeStruct((NC, NP, 128), jnp.float32),
    scratch_types=[
        pltpu.VMEM((80, 64), jnp.int32),
        pltpu.VMEM((80, 64), jnp.int32),
        pltpu.VMEM((4, 64, 128), jnp.float32),
        pltpu.VMEM_SHARED((NP, 128), jnp.float32),
        pltpu.SemaphoreType.DMA,
        pltpu.SemaphoreType.DMA,
        pltpu.SemaphoreType.DMA,
        pltpu.SemaphoreType.DMA,
    ],
)
def _agg1(y0, y1, src3, dst3, out_hbm, srcv, dstv, rows, acc,
          g0, g1, g2, g3):
    c = lax.axis_index("c")
    s = lax.axis_index("s")
    nch = EP // NS // 64
    half = nch // 2

    zero16 = _Z16()

    def z(i, _):
        rows[0, i // 8, pl.ds((i % 8) * 16, 16)] = zero16
        return 0
    lax.fori_loop(0, 512, z, 0)

    nrows = NP // NS

    def zacc(t, _):
        pltpu.sync_copy(rows.at[0], acc.at[pl.ds(s * nrows + t * 64, 64)])
        return 0
    lax.fori_loop(0, nrows // 64, zacc, 0)
    plsc.subcore_barrier()

    def run_edges(y_hbm):
        gsem = (g0, g1, g2, g3)
        nbuf = 4

        def wait_g(b):
            pltpu.make_async_copy(y_hbm.at[srcv.at[0]],
                                  rows.at[b], gsem[b]).wait()

        for h in range(2):
            base = s * nch + h * half
            pltpu.sync_copy(src3.at[pl.ds(base, half)], srcv)
            pltpu.sync_copy(dst3.at[pl.ds(base, half)], dstv)
            for b in range(nbuf):
                pltpu.async_copy(y_hbm.at[srcv.at[b]], rows.at[b], gsem[b])

            def step(g, _):
                for b in range(nbuf):
                    j = g * nbuf + b
                    wait_g(b)
                    pltpu.sync_copy(rows.at[b], acc.at[dstv.at[j]], add=True)

                    @pl.when(j + nbuf < half)
                    def _():
                        pltpu.async_copy(y_hbm.at[srcv.at[j + nbuf]],
                                         rows.at[b], gsem[b])
                return 0
            lax.fori_loop(0, half // nbuf, step, 0)

    @pl.when(c == 0)
    def _():
        run_edges(y0)

    @pl.when(c == 1)
    def _():
        run_edges(y1)

    plsc.subcore_barrier()

    def wb(t, _):
        pltpu.sync_copy(acc.at[pl.ds(s * nrows + t * 64, 64)], rows.at[0])
        pltpu.sync_copy(rows.at[0], out_hbm.at[c, pl.ds(s * nrows + t * 64, 64)])
        return 0
    lax.fori_loop(0, nrows // 64, wb, 0)


@functools.partial(
    pl.kernel, mesh=_mesh, compiler_params=_sc_params,
    out_type=jax.ShapeDtypeStruct((NC, NP, 16), jnp.float32),
    scratch_types=[
        pltpu.VMEM((EPT // 128, 128), jnp.int32),
        pltpu.VMEM((EPT // 128, 128), jnp.int32),
        pltpu.VMEM((4, 128, 16), jnp.float32),
        pltpu.VMEM_SHARED((NP, 16), jnp.float32),
        pltpu.SemaphoreType.DMA,
        pltpu.SemaphoreType.DMA,
        pltpu.SemaphoreType.DMA,
        pltpu.SemaphoreType.DMA,
    ],
)
def _agg2(q, src2, dst2, out_hbm, srcv, dstv, rows, acc,
          g0, g1, g2, g3):
    c = lax.axis_index("c")
    s = lax.axis_index("s")
    w = c * NS + s
    nch = EPT // 128
    pltpu.sync_copy(src2.at[pl.ds(w * nch, nch)], srcv)
    pltpu.sync_copy(dst2.at[pl.ds(w * nch, nch)], dstv)

    zero16 = _Z16()

    def z(j, _):
        rows[0, j, pl.ds(0, 16)] = zero16
        return 0
    lax.fori_loop(0, 128, z, 0)

    nrows = NP // NS

    def zacc(t, _):
        pltpu.sync_copy(rows.at[0], acc.at[pl.ds(s * nrows + t * 128, 128)])
        return 0
    lax.fori_loop(0, nrows // 128, zacc, 0)
    plsc.subcore_barrier()

    gsem = (g0, g1, g2, g3)
    nbuf = 4

    def wait_g(b):
        pltpu.make_async_copy(q.at[srcv.at[0]], rows.at[b], gsem[b]).wait()

    for b in range(nbuf):
        pltpu.async_copy(q.at[srcv.at[b]], rows.at[b], gsem[b])

    def step(g, _):
        for b in range(nbuf):
            j = g * nbuf + b
            wait_g(b)
            pltpu.sync_copy(rows.at[b], acc.at[dstv.at[j]], add=True)

            @pl.when(j + nbuf < nch)
            def _():
                pltpu.async_copy(q.at[srcv.at[j + nbuf]], rows.at[b], gsem[b])
        return 0
    lax.fori_loop(0, nch // nbuf, step, 0)

    plsc.subcore_barrier()

    def wb(t, _):
        pltpu.sync_copy(acc.at[pl.ds(s * nrows + t * 128, 128)], rows.at[0])
        pltpu.sync_copy(rows.at[0], out_hbm.at[c, pl.ds(s * nrows + t * 128, 128)])
        return 0
    lax.fori_loop(0, nrows // 128, wb, 0)


def _k1_body(x_ref, w_ref, cnt_ref, y_ref, dinv_ref):
    cnt = cnt_ref[...]
    dinv = lax.rsqrt(jnp.maximum(cnt[0] + cnt[1], 1.0))
    dinv_ref[...] = dinv
    y = jnp.dot(x_ref[...], w_ref[...], preferred_element_type=jnp.float32)
    y_ref[0] = y * dinv[0][:, None]


def _k1(x_p, W1, counts):
    return pl.pallas_call(
        _k1_body,
        grid=(2, NP // BR),
        in_specs=[
            pl.BlockSpec((BR, 256), lambda c, r: (r, 0)),
            pl.BlockSpec((256, 128), lambda c, r: (0, c)),
            pl.BlockSpec((2, 2, BR), lambda c, r: (0, 0, r)),
        ],
        out_specs=[
            pl.BlockSpec((1, BR, 128), lambda c, r: (c, r, 0)),
            pl.BlockSpec((2, BR), lambda c, r: (0, r)),
        ],
        out_shape=[
            jax.ShapeDtypeStruct((2, NP, 128), jnp.float32),
            jax.ShapeDtypeStruct((2, NP), jnp.float32),
        ],
    )(x_p, W1, counts)


def _k2_body(agg_ref, dinv_ref, b1_ref, w4t_ref, w4b_ref, w2_ref, et_ref,
             b2_ref, idx_ref, q_ref):
    a = agg_ref[...]
    dinv = dinv_ref[...]
    aggf = jnp.concatenate([a[0], a[1]], axis=1)
    h = jnp.maximum(aggf * dinv[1][:, None] + b1_ref[...], 0.0)
    acc = jnp.dot(h, w4t_ref[...], preferred_element_type=jnp.float32)
    M = jnp.dot(w2_ref[...], w4b_ref[...],
                preferred_element_type=jnp.float32).reshape(4, 256, 16)
    idx = idx_ref[...]
    et = et_ref[...]
    for l in range(4):
        g = jnp.dot(et, M[l], preferred_element_type=jnp.float32)
        oh = (idx[:, l][:, None]
              == lax.broadcasted_iota(jnp.int32, (BR, 256), 1)).astype(jnp.float32)
        acc = acc + jnp.dot(oh, g, preferred_element_type=jnp.float32)
    c0 = jnp.dot(b2_ref[...], w4b_ref[...], preferred_element_type=jnp.float32)
    q_ref[...] = (acc + c0) * dinv[0][:, None]


def _k2(agg, dinv, b1, W4t, W4b, W2, et, b2, idx_p):
    return pl.pallas_call(
        _k2_body,
        grid=(NP // BR,),
        in_specs=[
            pl.BlockSpec((2, BR, 128), lambda r: (0, r, 0)),
            pl.BlockSpec((2, BR), lambda r: (0, r)),
            pl.BlockSpec((1, 256), lambda r: (0, 0)),
            pl.BlockSpec((256, 16), lambda r: (0, 0)),
            pl.BlockSpec((2, 16), lambda r: (0, 0)),
            pl.BlockSpec((1024, 2), lambda r: (0, 0)),
            pl.BlockSpec((256, 256), lambda r: (0, 0)),
            pl.BlockSpec((1, 2), lambda r: (0, 0)),
            pl.BlockSpec((BR, 4), lambda r: (r, 0)),
        ],
        out_specs=pl.BlockSpec((BR, 16), lambda r: (r, 0)),
        out_shape=jax.ShapeDtypeStruct((NP, 16), jnp.float32),
    )(agg, dinv, b1, W4t, W4b, W2, et, b2, idx_p)


def _k3_body(p_ref, dinv_ref, b4_ref, o_ref):
    p = p_ref[...]
    o_ref[...] = (p[0] + p[1]) * dinv_ref[...][1][:, None] + b4_ref[...]


def _k3(parts, dinv, b4):
    return pl.pallas_call(
        _k3_body,
        grid=(NP // BR,),
        in_specs=[
            pl.BlockSpec((2, BR, 16), lambda r: (0, r, 0)),
            pl.BlockSpec((2, BR), lambda r: (0, r)),
            pl.BlockSpec((1, 16), lambda r: (0, 0)),
        ],
        out_specs=pl.BlockSpec((BR, 16), lambda r: (r, 0)),
        out_shape=jax.ShapeDtypeStruct((NP, 16), jnp.float32),
    )(parts, dinv, b4)


def kernel(in_feat, edge_index, encoder_input_data, W1, b1, embed_table,
           W2, b2, W4, b4):
    ei = edge_index.astype(jnp.int32)
    padv = jnp.full((EP - E,), N, jnp.int32)
    src_p = jnp.concatenate([ei[0], padv])
    dst_p = jnp.concatenate([ei[1], padv])
    src2 = src_p.reshape(EP // 128, 128)
    dst2 = dst_p.reshape(EP // 128, 128)
    x_p = jnp.pad(in_feat, ((0, NP - N), (0, 0)))
    idx_p = jnp.pad(encoder_input_data.astype(jnp.int32), ((0, NP - N), (0, 0)))

    src3 = src_p.reshape(EP // 64, 64)
    dst3 = dst_p.reshape(EP // 64, 64)

    counts = _deg(src_p, dst_p)
    y3, dinv = _k1(x_p, W1, counts)
    agg = _agg1(y3[0], y3[1], src3, dst3)
    q = _k2(agg, dinv, b1.reshape(1, 256), W4[:256], W4[256:], W2,
            embed_table, b2.reshape(1, 2), idx_p)
    parts = _agg2(q, src2, dst2)
    out = _k3(parts, dinv, b4.reshape(1, 16))
    return out[:N]
